# baseline (speedup 1.0000x reference)
import functools

import jax
import jax.numpy as jnp
from jax import lax
from jax.experimental import pallas as pl
from jax.experimental.pallas import tpu as pltpu
from jax.experimental.pallas import tpu_sc as plsc

B = 16384
D = 64
W = 2048
OF = 249856
NBLK = 123
YROWS = NBLK * W
NC = 2
NS = 16
NW = NC * NS
BPW = B // NW
CHUNK = 128
NCHUNK = BPW // CHUNK


def _bf16_hi_bits(x):
    r = lax.bitcast_convert_type(x, jnp.int32)
    r = r + jnp.int32(0x7FFF) + (lax.shift_right_logical(r, 16) & 1)
    return lax.shift_right_logical(r, 16)


def _tc_convert(tT):
    def body(a_ref, b_ref, c_ref, d_ref, o_ref):
        lo = jnp.concatenate([a_ref[...], c_ref[...]], axis=0).T
        hi = jnp.concatenate([b_ref[...], d_ref[...]], axis=0).T
        o_ref[...] = _bf16_hi_bits(lo) | lax.shift_left(_bf16_hi_bits(hi), 16)

    return pl.pallas_call(
        body,
        grid=(NBLK,),
        in_specs=[
            pl.BlockSpec((D, W), lambda j: (0, j)),
            pl.BlockSpec((D, W), lambda j: (0, 122 + j)),
            pl.BlockSpec((D, W), lambda j: (0, 244 + j)),
            pl.BlockSpec((D, W), lambda j: (0, 366 + j)),
        ],
        out_specs=pl.BlockSpec((W, 2 * D), lambda j: (j, 0)),
        out_shape=jax.ShapeDtypeStruct((YROWS, 2 * D), jnp.int32),
        compiler_params=pltpu.CompilerParams(
            dimension_semantics=("parallel",)),
    )(tT, tT, tT, tT)


def _sc_gather(idx2d, table2):
    mesh = plsc.VectorSubcoreMesh(core_axis_name="c", subcore_axis_name="s")

    @functools.partial(
        pl.kernel,
        mesh=mesh,
        out_type=jax.ShapeDtypeStruct((B, 2 * D), jnp.int32),
        scratch_types=[
            pltpu.VMEM((NCHUNK, CHUNK), jnp.int32),
            pltpu.VMEM((BPW, 2 * D), jnp.int32),
            pltpu.SemaphoreType.DMA,
        ],
    )
    def k(ix_hbm, t_hbm, o_hbm, ix_v, rows_v, sem):
        wid = lax.axis_index("s") * NC + lax.axis_index("c")
        base = wid * BPW
        pltpu.sync_copy(ix_hbm.at[pl.ds(wid * NCHUNK, NCHUNK)], ix_v)
        copies = []
        for g in range(NCHUNK):
            copies.append(pltpu.async_copy(
                t_hbm.at[ix_v.at[g]],
                rows_v.at[pl.ds(g * CHUNK, CHUNK)], sem))
        for cp in copies:
            cp.wait()
        pltpu.sync_copy(rows_v, o_hbm.at[pl.ds(base, BPW)])

    return k(idx2d, table2)


def _unpack(x_ref, idx_ref):
    k = jnp.minimum(idx_ref[...] // OF, 3)[:, None]
    x = x_ref[...]
    s = jnp.where(k >= 2, x[:, D:], x[:, :D])
    bits = jnp.where((k & 1) == 1,
                     lax.shift_right_logical(s, 16) & jnp.int32(0xFFFF),
                     s & jnp.int32(0xFFFF))
    return lax.bitcast_convert_type(lax.shift_left(bits, 16), jnp.float32)


def _tc_cosine(qg, cg, user, item):
    def body(q_ref, c_ref, u_ref, i_ref, o_ref):
        qv = _unpack(q_ref, u_ref)
        cv = _unpack(c_ref, i_ref)
        eps = jnp.float32(1e-8)
        dot = jnp.sum(qv * cv, axis=-1)
        qn = jnp.maximum(jnp.sqrt(jnp.sum(qv * qv, axis=-1)), eps)
        cn = jnp.maximum(jnp.sqrt(jnp.sum(cv * cv, axis=-1)), eps)
        o_ref[...] = dot / (qn * cn)

    return pl.pallas_call(
        body,
        out_shape=jax.ShapeDtypeStruct((B,), jnp.float32),
    )(qg, cg, user, item)


def kernel(user, item, user_table, item_table):
    yu = _tc_convert(user_table.T)
    yi = _tc_convert(item_table.T)
    uq = jnp.minimum(user // OF, 3)
    iq = jnp.minimum(item // OF, 3)
    uix = (user - uq * OF).reshape(NW * NCHUNK, CHUNK)
    iix = (item - iq * OF).reshape(NW * NCHUNK, CHUNK)
    qg = _sc_gather(uix, yu)
    cg = _sc_gather(iix, yi)
    return _tc_cosine(qg, cg, user, item)


# speedup vs baseline: 1.1998x; 1.1998x over previous
import functools

import jax
import jax.numpy as jnp
from jax import lax
from jax.experimental import pallas as pl
from jax.experimental.pallas import tpu as pltpu
from jax.experimental.pallas import tpu_sc as plsc

B = 16384
D = 64
W = 4096
OF = 249856
NBLK = 62
YROWS = NBLK * W
NC = 2
NS = 16
NW = NC * NS
BPW = B // NW
CHUNK = 128
NCHUNK = BPW // CHUNK


def _bf16_hi_bits(x):
    r = lax.bitcast_convert_type(x, jnp.int32)
    r = r + jnp.int32(0x7FFF) + (lax.shift_right_logical(r, 16) & 1)
    return lax.shift_right_logical(r, 16)


def _tc_convert(tT):
    def body(a_ref, b_ref, c_ref, d_ref, o_ref):
        lo = jnp.concatenate([a_ref[...], c_ref[...]], axis=0).T
        hi = jnp.concatenate([b_ref[...], d_ref[...]], axis=0).T
        o_ref[...] = _bf16_hi_bits(lo) | lax.shift_left(_bf16_hi_bits(hi), 16)

    return pl.pallas_call(
        body,
        grid=(NBLK,),
        in_specs=[
            pl.BlockSpec((D, W), lambda j: (0, j)),
            pl.BlockSpec((D, W), lambda j: (0, 61 + j)),
            pl.BlockSpec((D, W), lambda j: (0, 122 + j)),
            pl.BlockSpec((D, W), lambda j: (0, 183 + j)),
        ],
        out_specs=pl.BlockSpec((W, 2 * D), lambda j: (j, 0)),
        out_shape=jax.ShapeDtypeStruct((YROWS, 2 * D), jnp.int32),
        compiler_params=pltpu.CompilerParams(
            dimension_semantics=("parallel",)),
    )(tT, tT, tT, tT)


def _sc_gather(idx2d, table2):
    mesh = plsc.VectorSubcoreMesh(core_axis_name="c", subcore_axis_name="s")

    @functools.partial(
        pl.kernel,
        mesh=mesh,
        out_type=jax.ShapeDtypeStruct((B, 2 * D), jnp.int32),
        scratch_types=[
            pltpu.VMEM((NCHUNK, CHUNK), jnp.int32),
            pltpu.VMEM((BPW, 2 * D), jnp.int32),
            pltpu.SemaphoreType.DMA,
        ],
    )
    def k(ix_hbm, t_hbm, o_hbm, ix_v, rows_v, sem):
        wid = lax.axis_index("s") * NC + lax.axis_index("c")
        base = wid * BPW
        pltpu.sync_copy(ix_hbm.at[pl.ds(wid * NCHUNK, NCHUNK)], ix_v)
        copies = []
        for g in range(NCHUNK):
            copies.append(pltpu.async_copy(
                t_hbm.at[ix_v.at[g]],
                rows_v.at[pl.ds(g * CHUNK, CHUNK)], sem))
        for cp in copies:
            cp.wait()
        pltpu.sync_copy(rows_v, o_hbm.at[pl.ds(base, BPW)])

    return k(idx2d, table2)


def _unpack(x_ref, idx_ref):
    k = jnp.minimum(idx_ref[...] // OF, 3)[:, None]
    x = x_ref[...]
    s = jnp.where(k >= 2, x[:, D:], x[:, :D])
    bits = jnp.where((k & 1) == 1,
                     lax.shift_right_logical(s, 16) & jnp.int32(0xFFFF),
                     s & jnp.int32(0xFFFF))
    return lax.bitcast_convert_type(lax.shift_left(bits, 16), jnp.float32)


def _tc_cosine(qg, cg, user, item):
    def body(q_ref, c_ref, u_ref, i_ref, o_ref):
        qv = _unpack(q_ref, u_ref)
        cv = _unpack(c_ref, i_ref)
        eps = jnp.float32(1e-8)
        dot = jnp.sum(qv * cv, axis=-1)
        qn = jnp.maximum(jnp.sqrt(jnp.sum(qv * qv, axis=-1)), eps)
        cn = jnp.maximum(jnp.sqrt(jnp.sum(cv * cv, axis=-1)), eps)
        o_ref[...] = dot / (qn * cn)

    return pl.pallas_call(
        body,
        out_shape=jax.ShapeDtypeStruct((B,), jnp.float32),
    )(qg, cg, user, item)


def kernel(user, item, user_table, item_table):
    yu = _tc_convert(user_table.T)
    yi = _tc_convert(item_table.T)
    uq = jnp.minimum(user // OF, 3)
    iq = jnp.minimum(item // OF, 3)
    uix = (user - uq * OF).reshape(NW * NCHUNK, CHUNK)
    iix = (item - iq * OF).reshape(NW * NCHUNK, CHUNK)
    qg = _sc_gather(uix, yu)
    cg = _sc_gather(iix, yi)
    return _tc_cosine(qg, cg, user, item)


# speedup vs baseline: 1.2208x; 1.0175x over previous
import functools

import jax
import jax.numpy as jnp
from jax import lax
from jax.experimental import pallas as pl
from jax.experimental.pallas import tpu as pltpu
from jax.experimental.pallas import tpu_sc as plsc

B = 16384
D = 64
W = 4096
OF = 249856
NBLK = 62
YROWS = NBLK * W
NC = 2
NS = 16
NW = NC * NS
BPW = B // NW
CHUNK = 128
NCHUNK = BPW // CHUNK


def _bf16_hi_bits(x):
    r = lax.bitcast_convert_type(x, jnp.int32)
    r = r + jnp.int32(0x7FFF) + (lax.shift_right_logical(r, 16) & 1)
    return lax.shift_right_logical(r, 16)


def _tc_convert(tT):
    def body(a_ref, b_ref, c_ref, d_ref, o_ref):
        lo = jnp.concatenate([a_ref[...], c_ref[...]], axis=0).T
        hi = jnp.concatenate([b_ref[...], d_ref[...]], axis=0).T
        o_ref[...] = _bf16_hi_bits(lo) | lax.shift_left(_bf16_hi_bits(hi), 16)

    return pl.pallas_call(
        body,
        grid=(NBLK,),
        in_specs=[
            pl.BlockSpec((D, W), lambda j: (0, j)),
            pl.BlockSpec((D, W), lambda j: (0, 61 + j)),
            pl.BlockSpec((D, W), lambda j: (0, 122 + j)),
            pl.BlockSpec((D, W), lambda j: (0, 183 + j)),
        ],
        out_specs=pl.BlockSpec((W, 2 * D), lambda j: (j, 0)),
        out_shape=jax.ShapeDtypeStruct((YROWS, 2 * D), jnp.int32),
        compiler_params=pltpu.CompilerParams(
            dimension_semantics=("parallel",)),
    )(tT, tT, tT, tT)


def _sc_gather(idx2d, table2):
    mesh = plsc.VectorSubcoreMesh(core_axis_name="c", subcore_axis_name="s")

    @functools.partial(
        pl.kernel,
        mesh=mesh,
        out_type=jax.ShapeDtypeStruct((B, 2 * D), jnp.int32),
        scratch_types=[
            pltpu.VMEM((NCHUNK, CHUNK), jnp.int32),
            pltpu.VMEM((BPW, 2 * D), jnp.int32),
            pltpu.SemaphoreType.DMA,
        ],
    )
    def k(ix_hbm, t_hbm, o_hbm, ix_v, rows_v, sem):
        wid = lax.axis_index("s") * NC + lax.axis_index("c")
        base = wid * BPW
        pltpu.sync_copy(ix_hbm.at[pl.ds(wid * NCHUNK, NCHUNK)], ix_v)
        copies = []
        for g in range(NCHUNK):
            copies.append(pltpu.async_copy(
                t_hbm.at[ix_v.at[g]],
                rows_v.at[pl.ds(g * CHUNK, CHUNK)], sem))
        for cp in copies:
            cp.wait()
        pltpu.sync_copy(rows_v, o_hbm.at[pl.ds(base, BPW)])

    return k(idx2d, table2)


def _unpack(x_ref, idx_ref):
    u = idx_ref[...][:, None]
    hiwin = u >= 2 * OF
    odd = jnp.logical_xor(jnp.logical_xor(u >= OF, hiwin), u >= 3 * OF)
    x = x_ref[...]
    s = jnp.where(hiwin, x[:, D:], x[:, :D])
    bits = jnp.where(odd, lax.shift_right_logical(s, 16),
                     s & jnp.int32(0xFFFF))
    return lax.bitcast_convert_type(lax.shift_left(bits, 16), jnp.float32)


def _tc_cosine(qg, cg, user, item):
    def body(q_ref, c_ref, u_ref, i_ref, o_ref):
        qv = _unpack(q_ref, u_ref)
        cv = _unpack(c_ref, i_ref)
        eps = jnp.float32(1e-8)
        st = jnp.concatenate([qv * cv, qv * qv, cv * cv],
                             axis=0).astype(jnp.bfloat16)
        ones = jnp.ones((D, 8), jnp.bfloat16)
        sums = lax.dot_general(st, ones, (((1,), (0,)), ((), ())),
                               preferred_element_type=jnp.float32)
        dot = sums[0 * B:1 * B, 0]
        qn = jnp.maximum(jnp.sqrt(sums[1 * B:2 * B, 0]), eps)
        cn = jnp.maximum(jnp.sqrt(sums[2 * B:3 * B, 0]), eps)
        o_ref[...] = dot / (qn * cn)

    return pl.pallas_call(
        body,
        out_shape=jax.ShapeDtypeStruct((B,), jnp.float32),
    )(qg, cg, user, item)


def kernel(user, item, user_table, item_table):
    yu = _tc_convert(user_table.T)
    yi = _tc_convert(item_table.T)
    uq = ((user >= OF).astype(jnp.int32) + (user >= 2 * OF)
          + (user >= 3 * OF))
    iq = ((item >= OF).astype(jnp.int32) + (item >= 2 * OF)
          + (item >= 3 * OF))
    uix = (user - uq * OF).reshape(NW * NCHUNK, CHUNK)
    iix = (item - iq * OF).reshape(NW * NCHUNK, CHUNK)
    qg = _sc_gather(uix, yu)
    cg = _sc_gather(iix, yi)
    return _tc_cosine(qg, cg, user, item)
